# baseline (speedup 1.0000x reference)
import functools

import jax
import jax.numpy as jnp
from jax.experimental import pallas as pl
from jax.experimental.pallas import tpu as pltpu
from jax.experimental.pallas import tpu_sc as plsc

VPAD = 8


def _species_sc(species32, cat_emb):
    B = species32.shape[0]
    D = cat_emb.shape[1]
    mesh = plsc.VectorSubcoreMesh(core_axis_name="c", subcore_axis_name="s")

    @functools.partial(
        pl.kernel,
        out_type=jax.ShapeDtypeStruct((B, D), jnp.float32),
        mesh=mesh,
        scratch_types=[
            pltpu.VMEM((B,), jnp.int32),
            pltpu.VMEM((B, D), jnp.float32),
            pltpu.SemaphoreType.DMA,
        ],
    )
    def run(species_hbm, cat_hbm, out_hbm, idx_v, rows_v, sem):
        first = jnp.logical_and(
            jax.lax.axis_index("c") == 0, jax.lax.axis_index("s") == 0
        )

        @pl.when(first)
        def _():
            pltpu.sync_copy(species_hbm, idx_v)
            pltpu.async_copy(cat_hbm.at[idx_v], rows_v, sem).wait()
            pltpu.sync_copy(rows_v, out_hbm)

    return run(species32, cat_emb)


def _total_body(spe_idx_ref, seqs_ref, vocab_ref, cat_ref, pos_ref, out_ref):
    idx = seqs_ref[0, 0, :]
    blk = idx.shape[0]
    iota = jax.lax.broadcasted_iota(jnp.int32, (blk, VPAD), 1)
    oh = (idx[:, None] == iota).astype(jnp.float32)
    seq_emb = jnp.dot(oh, vocab_ref[...], preferred_element_type=jnp.float32)
    out_ref[...] = (seq_emb + cat_ref[0] + pos_ref[...])[None]


def kernel(seqs, species, vocab_emb, cat_emb, pos_emb):
    B, S = seqs.shape
    V, D = vocab_emb.shape
    blk = 2048
    NB = S // blk

    seqs3 = seqs.astype(jnp.int32).reshape(B * NB, 1, blk)
    species32 = species.astype(jnp.int32)
    vocab_pad = jnp.concatenate(
        [vocab_emb, jnp.zeros((VPAD - V, D), vocab_emb.dtype)], axis=0
    )
    cat3 = cat_emb.reshape(cat_emb.shape[0], 1, D)

    species_emb = _species_sc(species32, cat_emb)

    total = pl.pallas_call(
        _total_body,
        grid_spec=pltpu.PrefetchScalarGridSpec(
            num_scalar_prefetch=1,
            grid=(NB, B),
            in_specs=[
                pl.BlockSpec((1, 1, blk), lambda j, b, spe: (b * NB + j, 0, 0)),
                pl.BlockSpec((VPAD, D), lambda j, b, spe: (0, 0)),
                pl.BlockSpec((1, 1, D), lambda j, b, spe: (spe[b], 0, 0)),
                pl.BlockSpec((blk, D), lambda j, b, spe: (j, 0)),
            ],
            out_specs=pl.BlockSpec((1, blk, D), lambda j, b, spe: (b, j, 0)),
        ),
        out_shape=jax.ShapeDtypeStruct((B, S, D), jnp.float32),
        compiler_params=pltpu.CompilerParams(
            dimension_semantics=("arbitrary", "arbitrary")
        ),
    )(species32, seqs3, vocab_pad, cat3, pos_emb)

    return (total, species_emb)


# speedup vs baseline: 1.4120x; 1.4120x over previous
import jax
import jax.numpy as jnp
from jax.experimental import pallas as pl
from jax.experimental.pallas import tpu as pltpu

VPAD = 8


def _body(spe_idx_ref, seqs_ref, vocab_ref, cat_ref, pos_ref, out_ref, spe_out_ref):
    idx = seqs_ref[0, 0, :]
    n = idx.shape[0]
    iota = jax.lax.broadcasted_iota(jnp.int32, (n, VPAD), 1)
    oh = (idx[:, None] == iota).astype(jnp.float32)
    seq_emb = jnp.dot(oh, vocab_ref[...], preferred_element_type=jnp.float32)
    out_ref[...] = (seq_emb + cat_ref[0] + pos_ref[...])[None]
    spe_out_ref[...] = cat_ref[...]


def kernel(seqs, species, vocab_emb, cat_emb, pos_emb):
    B, S = seqs.shape
    V, D = vocab_emb.shape

    seqs3 = seqs.astype(jnp.int32).reshape(B, 1, S)
    species32 = species.astype(jnp.int32)
    vocab_pad = jnp.concatenate(
        [vocab_emb, jnp.zeros((VPAD - V, D), vocab_emb.dtype)], axis=0
    )
    cat3 = cat_emb.reshape(cat_emb.shape[0], 1, D)

    total, species_emb3 = pl.pallas_call(
        _body,
        grid_spec=pltpu.PrefetchScalarGridSpec(
            num_scalar_prefetch=1,
            grid=(B,),
            in_specs=[
                pl.BlockSpec((1, 1, S), lambda b, spe: (b, 0, 0)),
                pl.BlockSpec((VPAD, D), lambda b, spe: (0, 0)),
                pl.BlockSpec((1, 1, D), lambda b, spe: (spe[b], 0, 0)),
                pl.BlockSpec((S, D), lambda b, spe: (0, 0)),
            ],
            out_specs=[
                pl.BlockSpec((1, S, D), lambda b, spe: (b, 0, 0)),
                pl.BlockSpec((1, 1, D), lambda b, spe: (b, 0, 0)),
            ],
        ),
        out_shape=[
            jax.ShapeDtypeStruct((B, S, D), jnp.float32),
            jax.ShapeDtypeStruct((B, 1, D), jnp.float32),
        ],
        compiler_params=pltpu.CompilerParams(dimension_semantics=("arbitrary",)),
    )(species32, seqs3, vocab_pad, cat3, pos_emb)

    return (total, species_emb3.reshape(B, D))
